# baseline (speedup 1.0000x reference)
import functools

import jax
import jax.numpy as jnp
from jax import lax
from jax.experimental import pallas as pl
from jax.experimental.pallas import tpu as pltpu
from jax.experimental.pallas import tpu_sc as plsc

N = 10000
E = 320000
D = 128

NC = 2
NS = 16
NW = NC * NS
EPW = E // NW
C = 80
NCHUNK = EPW // C
ND = 10
RPT = N // ND
RCH = 40
NRC = RPT // RCH
L = 16


def _seg_sum_sc(table, src, dst, zeros_rows, zeros_hist=None):
    mesh = plsc.VectorSubcoreMesh(core_axis_name="c", subcore_axis_name="s",
                                  num_cores=NC, num_subcores=NS)

    with_counts = zeros_hist is not None
    out_type = [jax.ShapeDtypeStruct((NC, N, D), jnp.float32)]
    scratch = [
        pltpu.VMEM((EPW,), jnp.int32),
        pltpu.VMEM((NCHUNK, C), jnp.int32),
        pltpu.VMEM((C, D), jnp.float32),
        pltpu.VMEM((C, D), jnp.float32),
        pltpu.VMEM_SHARED((N, D), jnp.float32),
        pltpu.SemaphoreType.DMA,
        pltpu.SemaphoreType.DMA,
    ]
    if with_counts:
        out_type.append(jax.ShapeDtypeStruct((NC, NS, N), jnp.float32))
        scratch[1] = pltpu.VMEM((EPW,), jnp.int32)
        scratch.insert(2, pltpu.VMEM((1, C), jnp.int32))
        scratch.insert(5, pltpu.VMEM((N,), jnp.float32))

    def body(table_hbm, src_hbm, dst_hbm, z_hbm, *refs):
        if with_counts:
            (zh_hbm, sums_hbm, cnts_hbm, sidx_v, didx_v, drow_v, rows0_v,
             rows1_v, hist_v, acc_sh, sem0, sem1) = refs
        else:
            (sums_hbm, sidx_v, didx_v, rows0_v, rows1_v,
             acc_sh, sem0, sem1) = refs
        cid = lax.axis_index("c")
        sid = lax.axis_index("s")
        wid = cid * NS + sid

        row0 = sid * RPT

        @pl.when(sid < ND)
        def _zero():
            pltpu.sync_copy(z_hbm, rows0_v.at[pl.ds(0, RCH)])

            def zstep(j, carry):
                pltpu.sync_copy(rows0_v.at[pl.ds(0, RCH)],
                                acc_sh.at[pl.ds(row0 + j * RCH, RCH)])
                return carry

            lax.fori_loop(0, NRC, zstep, 0)

        ebase = wid * EPW
        pltpu.sync_copy(src_hbm.at[pl.ds(ebase, EPW)], sidx_v)
        if with_counts:
            pltpu.sync_copy(dst_hbm.at[pl.ds(ebase, EPW)], didx_v)
            pltpu.sync_copy(zh_hbm, hist_v)
        else:
            pltpu.sync_copy(dst_hbm.at[wid], didx_v)
        plsc.subcore_barrier()
        ones_l = jnp.ones((L,), jnp.float32)

        def gather_start(i, buf, sem):
            return pltpu.async_copy(
                table_hbm.at[sidx_v.at[pl.ds(i * C, C)]], buf, sem)

        def gather_wait(buf, sem):
            pltpu.make_async_copy(
                table_hbm.at[sidx_v.at[pl.ds(0, C)]], buf, sem).wait()

        def consume(i, buf):
            if with_counts:
                for k in range(C // L):
                    drow_v[0, pl.ds(k * L, L)] = didx_v[pl.ds(i * C + k * L, L)]
                for k in range(C // L):
                    plsc.addupdate_scatter(hist_v, [drow_v[0, pl.ds(k * L, L)]],
                                           ones_l)
                dst_idx = drow_v.at[0]
            else:
                dst_idx = didx_v.at[i]
            pltpu.sync_copy(buf, acc_sh.at[dst_idx], add=True)

        gather_start(0, rows0_v, sem0)

        def step(j, carry):
            i0 = 2 * j
            gather_start(i0 + 1, rows1_v, sem1)
            gather_wait(rows0_v, sem0)
            consume(i0, rows0_v)
            gather_start(i0 + 2, rows0_v, sem0)
            gather_wait(rows1_v, sem1)
            consume(i0 + 1, rows1_v)
            return carry

        lax.fori_loop(0, (NCHUNK - 1) // 2, step, 0)
        gather_wait(rows0_v, sem0)
        consume(NCHUNK - 1, rows0_v)
        if with_counts:
            pltpu.sync_copy(hist_v, cnts_hbm.at[cid, sid])
        plsc.subcore_barrier()

        @pl.when(sid < ND)
        def _drain():
            def dstep(j, carry):
                r = row0 + j * RCH
                pltpu.sync_copy(acc_sh.at[pl.ds(r, RCH)], rows0_v.at[pl.ds(0, RCH)])
                pltpu.sync_copy(rows0_v.at[pl.ds(0, RCH)],
                                sums_hbm.at[cid, pl.ds(r, RCH)])
                return carry

            lax.fori_loop(0, NRC, dstep, 0)

    k = pl.kernel(body, out_type=out_type, mesh=mesh, scratch_types=scratch,
                  compiler_params=pltpu.CompilerParams(needs_layout_passes=False))
    if with_counts:
        return tuple(k(table, src, dst, zeros_rows, zeros_hist))
    return k(table, src, dst, zeros_rows)[0]


def _counts_sc(dst, zeros_hist):
    mesh = plsc.VectorSubcoreMesh(core_axis_name="c", subcore_axis_name="s",
                                  num_cores=NC, num_subcores=NS)

    scratch = [
        pltpu.VMEM((EPW,), jnp.int32),
        pltpu.VMEM((N,), jnp.float32),
    ]

    def body(dst_hbm, zh_hbm, cnts_hbm, didx_v, hist_v):
        cid = lax.axis_index("c")
        sid = lax.axis_index("s")
        wid = cid * NS + sid

        pltpu.sync_copy(zh_hbm, hist_v)
        pltpu.sync_copy(dst_hbm.at[pl.ds(wid * EPW, EPW)], didx_v)

        ones_l = jnp.ones((L,), jnp.float32)

        def step(i, carry):
            for k in range(C // L):
                idx = didx_v[pl.ds(i * C + k * L, L)]
                plsc.addupdate_scatter(hist_v, [idx], ones_l)
            return carry

        lax.fori_loop(0, NCHUNK, step, 0)
        pltpu.sync_copy(hist_v, cnts_hbm.at[cid, sid])

    k = pl.kernel(body, out_type=jax.ShapeDtypeStruct((NC, NS, N), jnp.float32),
                  mesh=mesh, scratch_types=scratch,
                  compiler_params=pltpu.CompilerParams(needs_layout_passes=False))
    return k(dst, zeros_hist)


def _sage_tc_body(relu, sums_ref, cnts_ref, x_ref, wl_ref, wr_ref, bl_ref, o_ref):
    s = sums_ref[0] + sums_ref[1]
    c = jnp.sum(cnts_ref[...], axis=1, keepdims=True)
    mean = s / jnp.maximum(c, 1.0)
    y = lax.dot_general(mean, wl_ref[...], (((1,), (1,)), ((), ())),
                        preferred_element_type=jnp.float32,
                        precision=lax.Precision.HIGHEST)
    y += lax.dot_general(x_ref[...], wr_ref[...], (((1,), (1,)), ((), ())),
                         preferred_element_type=jnp.float32,
                         precision=lax.Precision.HIGHEST)
    y += bl_ref[...]
    o_ref[...] = jnp.maximum(y, 0.0) if relu else y


def _sage_tc(sums, cnts, x, Wl, Wr, bl, relu):
    BN = 2000
    grid = (N // BN,)
    return pl.pallas_call(
        functools.partial(_sage_tc_body, relu),
        grid=grid,
        in_specs=[
            pl.BlockSpec((NC, BN, D), lambda i: (0, i, 0)),
            pl.BlockSpec((BN, NW), lambda i: (i, 0)),
            pl.BlockSpec((BN, D), lambda i: (i, 0)),
            pl.BlockSpec((D, D), lambda i: (0, 0)),
            pl.BlockSpec((D, D), lambda i: (0, 0)),
            pl.BlockSpec((1, D), lambda i: (0, 0)),
        ],
        out_specs=pl.BlockSpec((BN, D), lambda i: (i, 0)),
        out_shape=jax.ShapeDtypeStruct((N, D), jnp.float32),
    )(sums, cnts, x, Wl, Wr, bl)


def kernel(x, edge_index, W1l, b1l, W1r, W2l, b2l, W2r):
    src = edge_index[0]
    dst = edge_index[1]
    zeros_rows = jnp.zeros((RCH, D), jnp.float32)
    zeros_hist = jnp.zeros((N,), jnp.float32)

    dst3 = dst.reshape(NW, NCHUNK, C)
    sums1, cnts = _seg_sum_sc(x, src, dst, zeros_rows, zeros_hist)
    cnts_t = cnts.reshape(NW, N).T
    h = _sage_tc(sums1, cnts_t, x, W1l, W1r, b1l.reshape(1, D), relu=True)
    sums2 = _seg_sum_sc(h, src, dst3, zeros_rows)
    out = _sage_tc(sums2, cnts_t, h, W2l, W2r, b2l.reshape(1, D), relu=False)
    return out


# speedup vs baseline: 1.0239x; 1.0239x over previous
import functools

import jax
import jax.numpy as jnp
from jax import lax
from jax.experimental import pallas as pl
from jax.experimental.pallas import tpu as pltpu
from jax.experimental.pallas import tpu_sc as plsc

N = 10000
E = 320000
D = 128

NC = 2
NS = 16
NW = NC * NS
EPW = E // NW
C = 80
NCHUNK = EPW // C
ND = 10
RPT = N // ND
RCH = 40
NRC = RPT // RCH
L = 16


def _seg_sum_sc(table, src, dst, zeros_rows, zeros_hist=None):
    mesh = plsc.VectorSubcoreMesh(core_axis_name="c", subcore_axis_name="s",
                                  num_cores=NC, num_subcores=NS)

    with_counts = zeros_hist is not None
    out_type = [jax.ShapeDtypeStruct((NC, N, D), jnp.float32)]
    scratch = [
        pltpu.VMEM((EPW,), jnp.int32),
        pltpu.VMEM((NCHUNK, C), jnp.int32),
        pltpu.VMEM((C, D), jnp.float32),
        pltpu.VMEM((C, D), jnp.float32),
        pltpu.VMEM_SHARED((N, D), jnp.float32),
        pltpu.SemaphoreType.DMA,
        pltpu.SemaphoreType.DMA,
    ]
    if with_counts:
        out_type.append(jax.ShapeDtypeStruct((NC, NS, N), jnp.float32))
        scratch[1] = pltpu.VMEM((EPW,), jnp.int32)
        scratch.insert(2, pltpu.VMEM((1, C), jnp.int32))
        scratch.insert(5, pltpu.VMEM((N,), jnp.float32))

    def body(table_hbm, src_hbm, dst_hbm, z_hbm, *refs):
        if with_counts:
            (zh_hbm, sums_hbm, cnts_hbm, sidx_v, didx_v, drow_v, rows0_v,
             rows1_v, hist_v, acc_sh, sem0, sem1) = refs
        else:
            (sums_hbm, sidx_v, didx_v, rows0_v, rows1_v,
             acc_sh, sem0, sem1) = refs
        cid = lax.axis_index("c")
        sid = lax.axis_index("s")
        wid = cid * NS + sid

        row0 = sid * RPT

        @pl.when(sid < ND)
        def _zero():
            pltpu.sync_copy(z_hbm, rows0_v.at[pl.ds(0, RCH)])

            def zstep(j, carry):
                pltpu.sync_copy(rows0_v.at[pl.ds(0, RCH)],
                                acc_sh.at[pl.ds(row0 + j * RCH, RCH)])
                return carry

            lax.fori_loop(0, NRC, zstep, 0)

        ebase = wid * EPW
        pltpu.sync_copy(src_hbm.at[pl.ds(ebase, EPW)], sidx_v)
        if with_counts:
            pltpu.sync_copy(dst_hbm.at[pl.ds(ebase, EPW)], didx_v)
            pltpu.sync_copy(zh_hbm, hist_v)
        else:
            pltpu.sync_copy(dst_hbm.at[wid], didx_v)
        plsc.subcore_barrier()
        ones_l = jnp.ones((L,), jnp.float32)

        def gather_start(i, buf, sem):
            return pltpu.async_copy(
                table_hbm.at[sidx_v.at[pl.ds(i * C, C)]], buf, sem)

        def gather_wait(buf, sem):
            pltpu.make_async_copy(
                table_hbm.at[sidx_v.at[pl.ds(0, C)]], buf, sem).wait()

        def consume(i, buf):
            if with_counts:
                for k in range(C // L):
                    drow_v[0, pl.ds(k * L, L)] = didx_v[pl.ds(i * C + k * L, L)]
                for k in range(C // L):
                    plsc.addupdate_scatter(hist_v, [drow_v[0, pl.ds(k * L, L)]],
                                           ones_l)
                dst_idx = drow_v.at[0]
            else:
                dst_idx = didx_v.at[i]
            pltpu.sync_copy(buf, acc_sh.at[dst_idx], add=True)

        gather_start(0, rows0_v, sem0)

        def step(j, carry):
            i0 = 2 * j
            gather_start(i0 + 1, rows1_v, sem1)
            gather_wait(rows0_v, sem0)
            consume(i0, rows0_v)
            gather_start(i0 + 2, rows0_v, sem0)
            gather_wait(rows1_v, sem1)
            consume(i0 + 1, rows1_v)
            return carry

        lax.fori_loop(0, (NCHUNK - 1) // 2, step, 0)
        gather_wait(rows0_v, sem0)
        consume(NCHUNK - 1, rows0_v)
        if with_counts:
            pltpu.sync_copy(hist_v, cnts_hbm.at[cid, sid])
        plsc.subcore_barrier()

        @pl.when(sid < ND)
        def _drain():
            def dstep(j, carry):
                r = row0 + j * RCH
                pltpu.sync_copy(acc_sh.at[pl.ds(r, RCH)], rows0_v.at[pl.ds(0, RCH)])
                pltpu.sync_copy(rows0_v.at[pl.ds(0, RCH)],
                                sums_hbm.at[cid, pl.ds(r, RCH)])
                return carry

            lax.fori_loop(0, NRC, dstep, 0)

    k = pl.kernel(body, out_type=out_type, mesh=mesh, scratch_types=scratch,
                  compiler_params=pltpu.CompilerParams(needs_layout_passes=False))
    if with_counts:
        return tuple(k(table, src, dst, zeros_rows, zeros_hist))
    return k(table, src, dst, zeros_rows)[0]


def _counts_sc(dst, zeros_hist):
    mesh = plsc.VectorSubcoreMesh(core_axis_name="c", subcore_axis_name="s",
                                  num_cores=NC, num_subcores=NS)

    scratch = [
        pltpu.VMEM((EPW,), jnp.int32),
        pltpu.VMEM((N,), jnp.float32),
    ]

    def body(dst_hbm, zh_hbm, cnts_hbm, didx_v, hist_v):
        cid = lax.axis_index("c")
        sid = lax.axis_index("s")
        wid = cid * NS + sid

        pltpu.sync_copy(zh_hbm, hist_v)
        pltpu.sync_copy(dst_hbm.at[pl.ds(wid * EPW, EPW)], didx_v)

        ones_l = jnp.ones((L,), jnp.float32)

        def step(i, carry):
            for k in range(C // L):
                idx = didx_v[pl.ds(i * C + k * L, L)]
                plsc.addupdate_scatter(hist_v, [idx], ones_l)
            return carry

        lax.fori_loop(0, NCHUNK, step, 0)
        pltpu.sync_copy(hist_v, cnts_hbm.at[cid, sid])

    k = pl.kernel(body, out_type=jax.ShapeDtypeStruct((NC, NS, N), jnp.float32),
                  mesh=mesh, scratch_types=scratch,
                  compiler_params=pltpu.CompilerParams(needs_layout_passes=False))
    return k(dst, zeros_hist)


def _sage_tc_body(relu, sums_ref, cnts_ref, x_ref, wl_ref, wr_ref, bl_ref, o_ref):
    s = sums_ref[0] + sums_ref[1]
    c = jnp.sum(cnts_ref[...], axis=1, keepdims=True)
    mean = s / jnp.maximum(c, 1.0)
    y = lax.dot_general(mean, wl_ref[...], (((1,), (1,)), ((), ())),
                        preferred_element_type=jnp.float32)
    y += lax.dot_general(x_ref[...], wr_ref[...], (((1,), (1,)), ((), ())),
                         preferred_element_type=jnp.float32)
    y += bl_ref[...]
    o_ref[...] = jnp.maximum(y, 0.0) if relu else y


def _sage_tc(sums, cnts, x, Wl, Wr, bl, relu):
    BN = 2000
    grid = (N // BN,)
    return pl.pallas_call(
        functools.partial(_sage_tc_body, relu),
        grid=grid,
        in_specs=[
            pl.BlockSpec((NC, BN, D), lambda i: (0, i, 0)),
            pl.BlockSpec((BN, NW), lambda i: (i, 0)),
            pl.BlockSpec((BN, D), lambda i: (i, 0)),
            pl.BlockSpec((D, D), lambda i: (0, 0)),
            pl.BlockSpec((D, D), lambda i: (0, 0)),
            pl.BlockSpec((1, D), lambda i: (0, 0)),
        ],
        out_specs=pl.BlockSpec((BN, D), lambda i: (i, 0)),
        out_shape=jax.ShapeDtypeStruct((N, D), jnp.float32),
    )(sums, cnts, x, Wl, Wr, bl)


def kernel(x, edge_index, W1l, b1l, W1r, W2l, b2l, W2r):
    src = edge_index[0]
    dst = edge_index[1]
    zeros_rows = jnp.zeros((RCH, D), jnp.float32)
    zeros_hist = jnp.zeros((N,), jnp.float32)

    dst3 = dst.reshape(NW, NCHUNK, C)
    sums1, cnts = _seg_sum_sc(x, src, dst, zeros_rows, zeros_hist)
    cnts_t = cnts.reshape(NW, N).T
    h = _sage_tc(sums1, cnts_t, x, W1l, W1r, b1l.reshape(1, D), relu=True)
    sums2 = _seg_sum_sc(h, src, dst3, zeros_rows)
    out = _sage_tc(sums2, cnts_t, h, W2l, W2r, b2l.reshape(1, D), relu=False)
    return out


# speedup vs baseline: 1.0832x; 1.0579x over previous
import functools

import jax
import jax.numpy as jnp
from jax import lax
from jax.experimental import pallas as pl
from jax.experimental.pallas import tpu as pltpu
from jax.experimental.pallas import tpu_sc as plsc

N = 10000
E = 320000
D = 128

NC = 2
NS = 16
NW = NC * NS
EPW = E // NW
C = 80
NCHUNK = EPW // C
CP = 64
NCP = EPW // CP
REM = EPW - NCP * CP
ND = 10
RPT = N // ND
RCH = 40
NRC = RPT // RCH
L = 16


def _seg_sum_sc(table, src, dst, zeros_rows):
    mesh = plsc.VectorSubcoreMesh(core_axis_name="c", subcore_axis_name="s",
                                  num_cores=NC, num_subcores=NS)

    scratch = [
        pltpu.VMEM((EPW,), jnp.int32),
        pltpu.VMEM((EPW,), jnp.int32),
        pltpu.VMEM((3, CP), jnp.int32),
        pltpu.VMEM((1, L), jnp.int32),
        pltpu.VMEM((CP, D), jnp.float32),
        pltpu.VMEM((CP, D), jnp.float32),
        pltpu.VMEM((CP, D), jnp.float32),
        pltpu.VMEM_SHARED((N, D), jnp.float32),
        pltpu.SemaphoreType.DMA,
        pltpu.SemaphoreType.DMA,
        pltpu.SemaphoreType.DMA,
        pltpu.SemaphoreType.DMA,
    ]

    def body(table_hbm, src_hbm, dst_hbm, z_hbm, sums_hbm,
             sidx_v, didx_v, drow_v, drem_v, b0_v, b1_v, b2_v, acc_sh,
             g0, g1, g2, ssem):
        bufs = (b0_v, b1_v, b2_v)
        gsems = (g0, g1, g2)
        cid = lax.axis_index("c")
        sid = lax.axis_index("s")
        wid = cid * NS + sid

        row0 = sid * RPT

        @pl.when(sid < ND)
        def _zero():
            pltpu.sync_copy(z_hbm, b0_v.at[pl.ds(0, RCH)])

            def zstep(j, carry):
                pltpu.sync_copy(b0_v.at[pl.ds(0, RCH)],
                                acc_sh.at[pl.ds(row0 + j * RCH, RCH)])
                return carry

            lax.fori_loop(0, NRC, zstep, 0)

        ebase = wid * EPW
        pltpu.sync_copy(src_hbm.at[pl.ds(ebase, EPW)], sidx_v)
        pltpu.sync_copy(dst_hbm.at[pl.ds(ebase, EPW)], didx_v)
        plsc.subcore_barrier()

        pltpu.async_copy(
            table_hbm.at[sidx_v.at[pl.ds(NCP * CP, REM)]],
            b0_v.at[pl.ds(0, REM)], g0).wait()
        drem_v[0, :] = didx_v[pl.ds(NCP * CP, REM)]
        pltpu.sync_copy(b0_v.at[pl.ds(0, REM)], acc_sh.at[drem_v.at[0]],
                        add=True)

        def gather_start(i, r):
            pltpu.async_copy(
                table_hbm.at[sidx_v.at[pl.ds(i * CP, CP)]], bufs[r], gsems[r])

        def gather_wait(r):
            pltpu.make_async_copy(
                table_hbm.at[sidx_v.at[pl.ds(0, CP)]], bufs[r], gsems[r]).wait()

        def scatter_start(i, r):
            for k in range(CP // L):
                drow_v[r, pl.ds(k * L, L)] = didx_v[pl.ds(i * CP + k * L, L)]
            pltpu.async_copy(bufs[r], acc_sh.at[drow_v.at[r]], ssem, add=True)

        def scatter_wait(r):
            pltpu.make_async_copy(bufs[r], acc_sh.at[drow_v.at[r]], ssem).wait()

        gather_start(0, 0)
        gather_start(1, 1)
        gather_wait(0)
        scatter_start(0, 0)
        gather_start(2, 2)

        def step(j, carry):
            for k in range(3):
                i = 3 * j + 1 + k
                r = (1 + k) % 3
                gather_wait(r)
                scatter_wait((r + 2) % 3)
                scatter_start(i, r)
                gather_start(i + 2, (r + 2) % 3)
            return carry

        lax.fori_loop(0, (NCP - 3) // 3, step, 0)
        for i in (NCP - 2, NCP - 1):
            r = i % 3
            gather_wait(r)
            scatter_wait((r + 2) % 3)
            scatter_start(i, r)
        scatter_wait((NCP - 1) % 3)
        plsc.subcore_barrier()

        @pl.when(sid < ND)
        def _drain():
            def dstep(j, carry):
                r = row0 + j * RCH
                pltpu.sync_copy(acc_sh.at[pl.ds(r, RCH)], b0_v.at[pl.ds(0, RCH)])
                pltpu.sync_copy(b0_v.at[pl.ds(0, RCH)],
                                sums_hbm.at[cid, pl.ds(r, RCH)])
                return carry

            lax.fori_loop(0, NRC, dstep, 0)

    k = pl.kernel(body, out_type=jax.ShapeDtypeStruct((NC, N, D), jnp.float32),
                  mesh=mesh, scratch_types=scratch,
                  compiler_params=pltpu.CompilerParams(needs_layout_passes=False))
    return k(table, src, dst, zeros_rows)


def _counts_sc(dst, zeros_hist):
    mesh = plsc.VectorSubcoreMesh(core_axis_name="c", subcore_axis_name="s",
                                  num_cores=NC, num_subcores=NS)

    scratch = [
        pltpu.VMEM((EPW,), jnp.int32),
        pltpu.VMEM((N,), jnp.float32),
    ]

    def body(dst_hbm, zh_hbm, cnts_hbm, didx_v, hist_v):
        cid = lax.axis_index("c")
        sid = lax.axis_index("s")
        wid = cid * NS + sid

        pltpu.sync_copy(zh_hbm, hist_v)
        pltpu.sync_copy(dst_hbm.at[pl.ds(wid * EPW, EPW)], didx_v)

        ones_l = jnp.ones((L,), jnp.float32)

        def step(i, carry):
            for k in range(C // L):
                idx = didx_v[pl.ds(i * C + k * L, L)]
                plsc.addupdate_scatter(hist_v, [idx], ones_l)
            return carry

        lax.fori_loop(0, NCHUNK, step, 0)
        pltpu.sync_copy(hist_v, cnts_hbm.at[cid, sid])

    k = pl.kernel(body, out_type=jax.ShapeDtypeStruct((NC, NS, N), jnp.float32),
                  mesh=mesh, scratch_types=scratch,
                  compiler_params=pltpu.CompilerParams(needs_layout_passes=False))
    return k(dst, zeros_hist)


def _sage_tc_body(relu, sums_ref, cnts_ref, x_ref, wl_ref, wr_ref, bl_ref, o_ref):
    s = sums_ref[0] + sums_ref[1]
    c = jnp.sum(cnts_ref[...], axis=1, keepdims=True)
    mean = s / jnp.maximum(c, 1.0)
    y = lax.dot_general(mean, wl_ref[...], (((1,), (1,)), ((), ())),
                        preferred_element_type=jnp.float32)
    y += lax.dot_general(x_ref[...], wr_ref[...], (((1,), (1,)), ((), ())),
                         preferred_element_type=jnp.float32)
    y += bl_ref[...]
    o_ref[...] = jnp.maximum(y, 0.0) if relu else y


def _sage_tc(sums, cnts, x, Wl, Wr, bl, relu):
    BN = 2000
    grid = (N // BN,)
    return pl.pallas_call(
        functools.partial(_sage_tc_body, relu),
        grid=grid,
        in_specs=[
            pl.BlockSpec((NC, BN, D), lambda i: (0, i, 0)),
            pl.BlockSpec((BN, NW), lambda i: (i, 0)),
            pl.BlockSpec((BN, D), lambda i: (i, 0)),
            pl.BlockSpec((D, D), lambda i: (0, 0)),
            pl.BlockSpec((D, D), lambda i: (0, 0)),
            pl.BlockSpec((1, D), lambda i: (0, 0)),
        ],
        out_specs=pl.BlockSpec((BN, D), lambda i: (i, 0)),
        out_shape=jax.ShapeDtypeStruct((N, D), jnp.float32),
    )(sums, cnts, x, Wl, Wr, bl)


def kernel(x, edge_index, W1l, b1l, W1r, W2l, b2l, W2r):
    src = edge_index[0]
    dst = edge_index[1]
    zeros_rows = jnp.zeros((RCH, D), jnp.float32)
    zeros_hist = jnp.zeros((N,), jnp.float32)

    cnts = _counts_sc(dst, zeros_hist)
    cnts_t = cnts.reshape(NW, N).T
    sums1 = _seg_sum_sc(x, src, dst, zeros_rows)
    h = _sage_tc(sums1, cnts_t, x, W1l, W1r, b1l.reshape(1, D), relu=True)
    sums2 = _seg_sum_sc(h, src, dst, zeros_rows)
    out = _sage_tc(sums2, cnts_t, h, W2l, W2r, b2l.reshape(1, D), relu=False)
    return out
